# baseline (speedup 1.0000x reference)
import functools

import jax
import jax.numpy as jnp
from jax import lax
from jax.experimental import pallas as pl
from jax.experimental.pallas import tpu as pltpu
from jax.experimental.pallas import tpu_sc as plsc

_EPS = 1e-5
_NC = 2
_NS = 16
_NW = _NC * _NS
_CH = 128


def _bn_relu(y, g, bt):
    m = jnp.mean(y, axis=0, keepdims=True)
    v = jnp.mean((y - m) ** 2, axis=0, keepdims=True)
    return jnp.maximum((y - m) * lax.rsqrt(v + _EPS) * g + bt, 0.0)


def _matT(a, w):
    return lax.dot_general(a, w, (((1,), (1,)), ((), ())),
                           preferred_element_type=jnp.float32)


def _enc_body(x_ref, w_ref, b_ref, g_ref, bt_ref, c_ref, h_ref, hp_ref):
    h = _bn_relu(_matT(x_ref[...], w_ref[...]) + b_ref[...], g_ref[...],
                 bt_ref[...])
    h_ref[...] = h
    hp_ref[...] = jnp.maximum(h + c_ref[...], 0.0)


def _layer_body(h_ref, agg_ref, w_ref, g_ref, bt_ref, c_ref, h_ref_o, hp_ref):
    n = h_ref.shape[0]
    ag = agg_ref[...]
    h = h_ref[...]
    u = h + ag[:n] + ag[n:]
    hn = _bn_relu(_matT(u, w_ref[...]), g_ref[...], bt_ref[...]) + h
    h_ref_o[...] = hn
    hp_ref[...] = jnp.maximum(hn + c_ref[...], 0.0)


def _final_body(h_ref, agg_ref, w_ref, g_ref, bt_ref, batch_ref, w1_ref,
                b1_ref, g1_ref, bt1_ref, w2_ref, b2_ref, out_ref):
    n = h_ref.shape[0]
    g = out_ref.shape[0]
    ag = agg_ref[...]
    h = h_ref[...]
    u = h + ag[:n] + ag[n:]
    hn = _bn_relu(_matT(u, w_ref[...]), g_ref[...], bt_ref[...]) + h
    onehot = (batch_ref[...] == lax.broadcasted_iota(jnp.int32, (n, g), 1)
              ).astype(jnp.float32)
    pooled = lax.dot_general(onehot, hn, (((0,), (0,)), ((), ())),
                             preferred_element_type=jnp.float32,
                             precision=lax.Precision.HIGHEST)
    o = _bn_relu(_matT(pooled, w1_ref[...]) + b1_ref[...], g1_ref[...],
                 bt1_ref[...])
    out_ref[...] = _matT(o, w2_ref[...]) + b2_ref[...]


@functools.lru_cache(maxsize=None)
def _make_edge_agg(n, d, e_pad):
    assert e_pad % (_NW * _CH) == 0 and n % _NS == 0
    epw = e_pad // _NW
    nch = epw // _CH
    assert nch % 8 == 0
    rpt = (n // _NS) // 8 * 8
    rlast = n - (_NS - 1) * rpt
    assert rlast % 8 == 0
    mesh = plsc.VectorSubcoreMesh(core_axis_name="c", subcore_axis_name="s")

    @functools.partial(
        pl.kernel,
        out_type=jax.ShapeDtypeStruct((2 * n, d), jnp.float32),
        mesh=mesh,
        scratch_types=[
            pltpu.VMEM_SHARED((n + 8, d), jnp.float32),
            pltpu.VMEM((nch, _CH), jnp.int32),
            pltpu.VMEM((_CH,), jnp.int32),
            pltpu.VMEM((_CH,), jnp.int32),
            pltpu.VMEM((_CH, d), jnp.float32),
            pltpu.VMEM((_CH, d), jnp.float32),
            pltpu.SemaphoreType.DMA,
            pltpu.SemaphoreType.DMA,
        ],
    )
    def edge_agg(hp_hbm, src2_hbm, dst_hbm, zero_hbm, out_hbm,
                 acc, src_v, dst0, dst1, rows0, rows1, sem0, sem1):
        c = lax.axis_index("c")
        s = lax.axis_index("s")
        wid = s * _NC + c
        pltpu.sync_copy(src2_hbm.at[pl.ds(wid * nch, nch)], src_v)
        row0 = pl.multiple_of(s * rpt, 8)
        @pl.when(s < _NS - 1)
        def _():
            pltpu.sync_copy(zero_hbm.at[pl.ds(0, rpt)],
                            acc.at[pl.ds(row0, rpt)])
        @pl.when(s == _NS - 1)
        def _():
            pltpu.sync_copy(zero_hbm, acc.at[pl.ds((_NS - 1) * rpt, rlast)])
        plsc.subcore_barrier()

        base = wid * epw

        def pair(j, carry):
            k0 = 2 * j
            ga = pltpu.async_copy(hp_hbm.at[src_v.at[k0]], rows0, sem0)
            gb = pltpu.async_copy(hp_hbm.at[src_v.at[k0 + 1]], rows1, sem1)
            pltpu.sync_copy(dst_hbm.at[pl.ds(base + k0 * _CH, _CH)], dst0)
            pltpu.sync_copy(dst_hbm.at[pl.ds(base + (k0 + 1) * _CH, _CH)],
                            dst1)
            ga.wait()
            pltpu.sync_copy(rows0, acc.at[dst0], add=True)
            gb.wait()
            pltpu.sync_copy(rows1, acc.at[dst1], add=True)
            return carry

        lax.fori_loop(0, nch // 2, pair, 0)
        plsc.subcore_barrier()
        ob = pl.multiple_of(c * n + row0, 8)
        @pl.when(s < _NS - 1)
        def _():
            pltpu.sync_copy(acc.at[pl.ds(row0, rpt)],
                            out_hbm.at[pl.ds(ob, rpt)])
        @pl.when(s == _NS - 1)
        def _():
            pltpu.sync_copy(acc.at[pl.ds((_NS - 1) * rpt, rlast)],
                            out_hbm.at[pl.ds(c * n + (_NS - 1) * rpt, rlast)])

    return edge_agg


def kernel(x, edge_index, batch, W_in, b_in, g_in, bt_in, We, be, ge, bte,
           Wc, gn, btn, W1, b1, g1, bt1, W2, b2):
    n, d = x.shape
    e = edge_index.shape[1]
    nlayers = Wc.shape[0]
    g = 64
    row = lambda v: v.reshape(1, d)

    npad = (-e) % (_NW * _CH * 8)
    e_pad = e + npad
    src2 = jnp.concatenate(
        [edge_index[0], jnp.zeros((npad,), jnp.int32)]).reshape(
            e_pad // _CH, _CH)
    dstp = jnp.concatenate(
        [edge_index[1], jnp.full((npad,), n, jnp.int32)])
    cs = jnp.maximum(bte, 0.0)
    rpt = (n // _NS) // 8 * 8
    zrows = jnp.zeros((n - (_NS - 1) * rpt, d), jnp.float32)

    sds = jax.ShapeDtypeStruct
    two_nd = [sds((n, d), jnp.float32), sds((n, d), jnp.float32)]
    h, hp = pl.pallas_call(_enc_body, out_shape=two_nd)(
        x, W_in, row(b_in), row(g_in), row(bt_in), row(cs[0]))

    edge_agg = _make_edge_agg(n, d, e_pad)
    for i in range(nlayers):
        aggp = edge_agg(hp, src2, dstp, zrows)
        if i + 1 < nlayers:
            h, hp = pl.pallas_call(_layer_body, out_shape=two_nd)(
                h, aggp, Wc[i], row(gn[i]), row(btn[i]), row(cs[i + 1]))
        else:
            out = pl.pallas_call(
                _final_body, out_shape=sds((g, d), jnp.float32))(
                    h, aggp, Wc[i], row(gn[i]), row(btn[i]),
                    batch.reshape(n, 1), W1, row(b1), row(g1), row(bt1),
                    W2, row(b2))
    return out


# speedup vs baseline: 1.0104x; 1.0104x over previous
import functools

import jax
import jax.numpy as jnp
from jax import lax
from jax.experimental import pallas as pl
from jax.experimental.pallas import tpu as pltpu
from jax.experimental.pallas import tpu_sc as plsc

_EPS = 1e-5
_NC = 2
_NS = 16
_NW = _NC * _NS
_CH = 128


def _bn_relu(y, g, bt):
    m = jnp.mean(y, axis=0, keepdims=True)
    v = jnp.mean((y - m) ** 2, axis=0, keepdims=True)
    return jnp.maximum((y - m) * lax.rsqrt(v + _EPS) * g + bt, 0.0)


def _matT(a, w):
    return lax.dot_general(a, w, (((1,), (1,)), ((), ())),
                           preferred_element_type=jnp.float32)


def _enc_body(x_ref, w_ref, b_ref, g_ref, bt_ref, c_ref, h_ref, hp_ref):
    h = _bn_relu(_matT(x_ref[...], w_ref[...]) + b_ref[...], g_ref[...],
                 bt_ref[...])
    h_ref[...] = h
    hp_ref[...] = jnp.maximum(h + c_ref[...], 0.0)


def _layer_body(h_ref, agg_ref, w_ref, g_ref, bt_ref, c_ref, h_ref_o, hp_ref):
    n = h_ref.shape[0]
    ag = agg_ref[...]
    h = h_ref[...]
    u = h + ag[:n] + ag[n:]
    hn = _bn_relu(_matT(u, w_ref[...]), g_ref[...], bt_ref[...]) + h
    h_ref_o[...] = hn
    hp_ref[...] = jnp.maximum(hn + c_ref[...], 0.0)


def _final_body(h_ref, agg_ref, w_ref, g_ref, bt_ref, batch_ref, w1_ref,
                b1_ref, g1_ref, bt1_ref, w2_ref, b2_ref, out_ref):
    n = h_ref.shape[0]
    g = out_ref.shape[0]
    ag = agg_ref[...]
    h = h_ref[...]
    u = h + ag[:n] + ag[n:]
    hn = _bn_relu(_matT(u, w_ref[...]), g_ref[...], bt_ref[...]) + h
    onehot = (batch_ref[...] == lax.broadcasted_iota(jnp.int32, (n, g), 1)
              ).astype(jnp.float32)
    pooled = lax.dot_general(onehot, hn, (((0,), (0,)), ((), ())),
                             preferred_element_type=jnp.float32,
                             precision=lax.Precision.HIGHEST)
    o = _bn_relu(_matT(pooled, w1_ref[...]) + b1_ref[...], g1_ref[...],
                 bt1_ref[...])
    out_ref[...] = _matT(o, w2_ref[...]) + b2_ref[...]


@functools.lru_cache(maxsize=None)
def _make_edge_agg(n, d, e_pad):
    assert e_pad % (_NW * _CH) == 0 and n % _NS == 0
    epw = e_pad // _NW
    nch = epw // _CH
    assert nch % 8 == 0
    rpt = (n // _NS) // 8 * 8
    rlast = n - (_NS - 1) * rpt
    assert rlast % 8 == 0
    mesh = plsc.VectorSubcoreMesh(core_axis_name="c", subcore_axis_name="s")

    @functools.partial(
        pl.kernel,
        out_type=jax.ShapeDtypeStruct((2 * n, d), jnp.float32),
        mesh=mesh,
        scratch_types=[
            pltpu.VMEM_SHARED((n + 8, d), jnp.float32),
            pltpu.VMEM((_CH,), jnp.int32),
            pltpu.VMEM((_CH,), jnp.int32),
            pltpu.VMEM((_CH,), jnp.int32),
            pltpu.VMEM((_CH,), jnp.int32),
            pltpu.VMEM((_CH, d), jnp.float32),
            pltpu.VMEM((_CH, d), jnp.float32),
            pltpu.SemaphoreType.DMA,
            pltpu.SemaphoreType.DMA,
        ],
    )
    def edge_agg(hp_hbm, src_hbm, dst_hbm, zero_hbm, out_hbm,
                 acc, src0, src1, dst0, dst1, rows0, rows1, sem0, sem1):
        c = lax.axis_index("c")
        s = lax.axis_index("s")
        wid = s * _NC + c
        row0 = pl.multiple_of(s * rpt, 8)
        @pl.when(s < _NS - 1)
        def _():
            pltpu.sync_copy(zero_hbm.at[pl.ds(0, rpt)],
                            acc.at[pl.ds(row0, rpt)])
        @pl.when(s == _NS - 1)
        def _():
            pltpu.sync_copy(zero_hbm, acc.at[pl.ds((_NS - 1) * rpt, rlast)])
        plsc.subcore_barrier()

        base = wid * epw

        def pair(j, carry):
            o0 = base + 2 * j * _CH
            pltpu.sync_copy(src_hbm.at[pl.ds(o0, _CH)], src0)
            ga = pltpu.async_copy(hp_hbm.at[src0], rows0, sem0)
            pltpu.sync_copy(src_hbm.at[pl.ds(o0 + _CH, _CH)], src1)
            gb = pltpu.async_copy(hp_hbm.at[src1], rows1, sem1)
            pltpu.sync_copy(dst_hbm.at[pl.ds(o0, _CH)], dst0)
            pltpu.sync_copy(dst_hbm.at[pl.ds(o0 + _CH, _CH)], dst1)
            ga.wait()
            pltpu.sync_copy(rows0, acc.at[dst0], add=True)
            gb.wait()
            pltpu.sync_copy(rows1, acc.at[dst1], add=True)
            return carry

        lax.fori_loop(0, nch // 2, pair, 0)
        plsc.subcore_barrier()
        ob = pl.multiple_of(c * n + row0, 8)
        @pl.when(s < _NS - 1)
        def _():
            pltpu.sync_copy(acc.at[pl.ds(row0, rpt)],
                            out_hbm.at[pl.ds(ob, rpt)])
        @pl.when(s == _NS - 1)
        def _():
            pltpu.sync_copy(acc.at[pl.ds((_NS - 1) * rpt, rlast)],
                            out_hbm.at[pl.ds(c * n + (_NS - 1) * rpt, rlast)])

    return edge_agg


def kernel(x, edge_index, batch, W_in, b_in, g_in, bt_in, We, be, ge, bte,
           Wc, gn, btn, W1, b1, g1, bt1, W2, b2):
    n, d = x.shape
    e = edge_index.shape[1]
    nlayers = Wc.shape[0]
    g = 64
    row = lambda v: v.reshape(1, d)

    npad = (-e) % (_NW * _CH * 8)
    e_pad = e + npad
    srcp = jnp.concatenate([edge_index[0], jnp.zeros((npad,), jnp.int32)])
    dstp = jnp.concatenate([edge_index[1], jnp.full((npad,), n, jnp.int32)])
    cs = jnp.maximum(bte, 0.0)
    rpt = (n // _NS) // 8 * 8
    zrows = jnp.zeros((n - (_NS - 1) * rpt, d), jnp.float32)

    sds = jax.ShapeDtypeStruct
    two_nd = [sds((n, d), jnp.float32), sds((n, d), jnp.float32)]
    h, hp = pl.pallas_call(_enc_body, out_shape=two_nd)(
        x, W_in, row(b_in), row(g_in), row(bt_in), row(cs[0]))

    edge_agg = _make_edge_agg(n, d, e_pad)
    for i in range(nlayers):
        aggp = edge_agg(hp, srcp, dstp, zrows)
        if i + 1 < nlayers:
            h, hp = pl.pallas_call(_layer_body, out_shape=two_nd)(
                h, aggp, Wc[i], row(gn[i]), row(btn[i]), row(cs[i + 1]))
        else:
            out = pl.pallas_call(
                _final_body, out_shape=sds((g, d), jnp.float32))(
                    h, aggp, Wc[i], row(gn[i]), row(btn[i]),
                    batch.reshape(n, 1), W1, row(b1), row(g1), row(bt1),
                    W2, row(b2))
    return out


# speedup vs baseline: 3.1628x; 3.1303x over previous
import functools

import jax
import jax.numpy as jnp
from jax import lax
from jax.experimental import pallas as pl
from jax.experimental.pallas import tpu as pltpu
from jax.experimental.pallas import tpu_sc as plsc

_EPS = 1e-5
_NC = 2
_NS = 16
_NW = _NC * _NS
_CH = 128


def _bn_relu(y, g, bt):
    m = jnp.mean(y, axis=0, keepdims=True)
    v = jnp.mean((y - m) ** 2, axis=0, keepdims=True)
    return jnp.maximum((y - m) * lax.rsqrt(v + _EPS) * g + bt, 0.0)


def _matT(a, w):
    return lax.dot_general(a, w, (((1,), (1,)), ((), ())),
                           preferred_element_type=jnp.float32)


def _enc_body(x_ref, w_ref, b_ref, g_ref, bt_ref, c_ref, h_ref, hp_ref):
    h = _bn_relu(_matT(x_ref[...], w_ref[...]) + b_ref[...], g_ref[...],
                 bt_ref[...])
    h_ref[...] = h
    hp_ref[...] = jnp.maximum(h + c_ref[...], 0.0)


def _layer_body(h_ref, agg_ref, w_ref, g_ref, bt_ref, c_ref, h_ref_o, hp_ref):
    n = h_ref.shape[0]
    ag = agg_ref[...]
    h = h_ref[...]
    u = h + ag[:n] + ag[n:]
    hn = _bn_relu(_matT(u, w_ref[...]), g_ref[...], bt_ref[...]) + h
    h_ref_o[...] = hn
    hp_ref[...] = jnp.maximum(hn + c_ref[...], 0.0)


def _final_body(h_ref, agg_ref, w_ref, g_ref, bt_ref, batch_ref, w1_ref,
                b1_ref, g1_ref, bt1_ref, w2_ref, b2_ref, out_ref):
    n = h_ref.shape[0]
    g = out_ref.shape[0]
    ag = agg_ref[...]
    h = h_ref[...]
    u = h + ag[:n] + ag[n:]
    hn = _bn_relu(_matT(u, w_ref[...]), g_ref[...], bt_ref[...]) + h
    onehot = (batch_ref[...] == lax.broadcasted_iota(jnp.int32, (n, g), 1)
              ).astype(jnp.float32)
    pooled = lax.dot_general(onehot, hn, (((0,), (0,)), ((), ())),
                             preferred_element_type=jnp.float32,
                             precision=lax.Precision.HIGHEST)
    o = _bn_relu(_matT(pooled, w1_ref[...]) + b1_ref[...], g1_ref[...],
                 bt1_ref[...])
    out_ref[...] = _matT(o, w2_ref[...]) + b2_ref[...]


@functools.lru_cache(maxsize=None)
def _make_edge_agg(n, d, e):
    assert e % _NW == 0 and n % _NS == 0
    epw = e // _NW
    nfull = epw // _CH
    npair = nfull // 2
    tail = epw - npair * 2 * _CH
    assert tail % 8 == 0 and tail <= _CH
    rpt = (n // _NS) // 8 * 8
    rlast = n - (_NS - 1) * rpt
    assert rlast % 8 == 0
    mesh = plsc.VectorSubcoreMesh(core_axis_name="c", subcore_axis_name="s")

    @functools.partial(
        pl.kernel,
        out_type=jax.ShapeDtypeStruct((2 * n, d), jnp.float32),
        mesh=mesh,
        scratch_types=[
            pltpu.VMEM_SHARED((n, d), jnp.float32),
            pltpu.VMEM((_CH,), jnp.int32),
            pltpu.VMEM((_CH,), jnp.int32),
            pltpu.VMEM((_CH,), jnp.int32),
            pltpu.VMEM((_CH,), jnp.int32),
            pltpu.VMEM((_CH, d), jnp.float32),
            pltpu.VMEM((_CH, d), jnp.float32),
            pltpu.VMEM((max(tail, 8),), jnp.int32),
            pltpu.VMEM((max(tail, 8),), jnp.int32),
            pltpu.VMEM((max(tail, 8), d), jnp.float32),
            pltpu.SemaphoreType.DMA,
            pltpu.SemaphoreType.DMA,
        ],
    )
    def edge_agg(hp_hbm, src_hbm, dst_hbm, zero_hbm, out_hbm,
                 acc, src0, src1, dst0, dst1, rows0, rows1,
                 src_t, dst_t, rows_t, sem0, sem1):
        c = lax.axis_index("c")
        s = lax.axis_index("s")
        wid = s * _NC + c
        row0 = pl.multiple_of(s * rpt, 8)
        @pl.when(s < _NS - 1)
        def _():
            pltpu.sync_copy(zero_hbm.at[pl.ds(0, rpt)],
                            acc.at[pl.ds(row0, rpt)])
        @pl.when(s == _NS - 1)
        def _():
            pltpu.sync_copy(zero_hbm, acc.at[pl.ds((_NS - 1) * rpt, rlast)])
        plsc.subcore_barrier()

        base = wid * epw

        def pair(j, carry):
            o0 = base + 2 * j * _CH
            pltpu.sync_copy(src_hbm.at[pl.ds(o0, _CH)], src0)
            ga = pltpu.async_copy(hp_hbm.at[src0], rows0, sem0)
            pltpu.sync_copy(src_hbm.at[pl.ds(o0 + _CH, _CH)], src1)
            gb = pltpu.async_copy(hp_hbm.at[src1], rows1, sem1)
            pltpu.sync_copy(dst_hbm.at[pl.ds(o0, _CH)], dst0)
            pltpu.sync_copy(dst_hbm.at[pl.ds(o0 + _CH, _CH)], dst1)
            ga.wait()
            pltpu.sync_copy(rows0, acc.at[dst0], add=True)
            gb.wait()
            pltpu.sync_copy(rows1, acc.at[dst1], add=True)
            return carry

        lax.fori_loop(0, npair, pair, 0)
        if tail:
            ot = base + npair * 2 * _CH
            pltpu.sync_copy(src_hbm.at[pl.ds(ot, tail)], src_t)
            gt = pltpu.async_copy(hp_hbm.at[src_t], rows_t, sem0)
            pltpu.sync_copy(dst_hbm.at[pl.ds(ot, tail)], dst_t)
            gt.wait()
            pltpu.sync_copy(rows_t, acc.at[dst_t], add=True)
        plsc.subcore_barrier()
        ob = pl.multiple_of(c * n + row0, 8)
        @pl.when(s < _NS - 1)
        def _():
            pltpu.sync_copy(acc.at[pl.ds(row0, rpt)],
                            out_hbm.at[pl.ds(ob, rpt)])
        @pl.when(s == _NS - 1)
        def _():
            pltpu.sync_copy(acc.at[pl.ds((_NS - 1) * rpt, rlast)],
                            out_hbm.at[pl.ds(c * n + (_NS - 1) * rpt, rlast)])

    return edge_agg


def kernel(x, edge_index, batch, W_in, b_in, g_in, bt_in, We, be, ge, bte,
           Wc, gn, btn, W1, b1, g1, bt1, W2, b2):
    n, d = x.shape
    e = edge_index.shape[1]
    nlayers = Wc.shape[0]
    g = 64
    row = lambda v: v.reshape(1, d)

    src = edge_index[0]
    dst = edge_index[1]
    cs = jnp.maximum(bte, 0.0)
    rpt = (n // _NS) // 8 * 8
    zrows = jnp.zeros((n - (_NS - 1) * rpt, d), jnp.float32)

    sds = jax.ShapeDtypeStruct
    two_nd = [sds((n, d), jnp.float32), sds((n, d), jnp.float32)]
    h, hp = pl.pallas_call(_enc_body, out_shape=two_nd)(
        x, W_in, row(b_in), row(g_in), row(bt_in), row(cs[0]))

    edge_agg = _make_edge_agg(n, d, e)
    for i in range(nlayers):
        aggp = edge_agg(hp, src, dst, zrows)
        if i + 1 < nlayers:
            h, hp = pl.pallas_call(_layer_body, out_shape=two_nd)(
                h, aggp, Wc[i], row(gn[i]), row(btn[i]), row(cs[i + 1]))
        else:
            out = pl.pallas_call(
                _final_body, out_shape=sds((g, d), jnp.float32))(
                    h, aggp, Wc[i], row(gn[i]), row(btn[i]),
                    batch.reshape(n, 1), W1, row(b1), row(g1), row(bt1),
                    W2, row(b2))
    return out


# speedup vs baseline: 3.1906x; 1.0088x over previous
import functools

import jax
import jax.numpy as jnp
from jax import lax
from jax.experimental import pallas as pl
from jax.experimental.pallas import tpu as pltpu
from jax.experimental.pallas import tpu_sc as plsc

_EPS = 1e-5
_NC = 2
_NS = 16
_NW = _NC * _NS
_CH = 128


def _bn_relu(y, g, bt):
    m = jnp.mean(y, axis=0, keepdims=True)
    v = jnp.mean((y - m) ** 2, axis=0, keepdims=True)
    return jnp.maximum((y - m) * lax.rsqrt(v + _EPS) * g + bt, 0.0)


def _matT(a, w):
    return lax.dot_general(a, w, (((1,), (1,)), ((), ())),
                           preferred_element_type=jnp.float32)


def _enc_body(x_ref, w_ref, b_ref, g_ref, bt_ref, c_ref, h_ref, hp_ref):
    h = _bn_relu(_matT(x_ref[...], w_ref[...]) + b_ref[...], g_ref[...],
                 bt_ref[...])
    h_ref[...] = h
    hp_ref[...] = jnp.maximum(h + c_ref[...], 0.0)


def _layer_body(h_ref, agg_ref, w_ref, g_ref, bt_ref, c_ref, h_ref_o, hp_ref):
    n = h_ref.shape[0]
    ag = agg_ref[...]
    h = h_ref[...]
    u = h + ag[:n] + ag[n:]
    hn = _bn_relu(_matT(u, w_ref[...]), g_ref[...], bt_ref[...]) + h
    h_ref_o[...] = hn
    hp_ref[...] = jnp.maximum(hn + c_ref[...], 0.0)


def _final_body(h_ref, agg_ref, w_ref, g_ref, bt_ref, batch_ref, w1_ref,
                b1_ref, g1_ref, bt1_ref, w2_ref, b2_ref, out_ref):
    n = h_ref.shape[0]
    g = out_ref.shape[0]
    ag = agg_ref[...]
    h = h_ref[...]
    u = h + ag[:n] + ag[n:]
    hn = _bn_relu(_matT(u, w_ref[...]), g_ref[...], bt_ref[...]) + h
    onehot = (batch_ref[...] == lax.broadcasted_iota(jnp.int32, (n, g), 1)
              ).astype(jnp.float32)
    pooled = lax.dot_general(onehot, hn, (((0,), (0,)), ((), ())),
                             preferred_element_type=jnp.float32,
                             precision=lax.Precision.HIGHEST)
    o = _bn_relu(_matT(pooled, w1_ref[...]) + b1_ref[...], g1_ref[...],
                 bt1_ref[...])
    out_ref[...] = _matT(o, w2_ref[...]) + b2_ref[...]


@functools.lru_cache(maxsize=None)
def _make_edge_agg(n, d, e):
    assert e % _NW == 0 and n % _NS == 0
    epw = e // _NW
    nfull = epw // _CH
    npair = nfull // 2
    tail = epw - npair * 2 * _CH
    assert tail % 8 == 0 and tail <= _CH
    rpt = (n // _NS) // 8 * 8
    rlast = n - (_NS - 1) * rpt
    assert rlast % 8 == 0
    mesh = plsc.VectorSubcoreMesh(core_axis_name="c", subcore_axis_name="s")

    @functools.partial(
        pl.kernel,
        out_type=jax.ShapeDtypeStruct((2 * n, d), jnp.float32),
        mesh=mesh,
        scratch_types=[
            pltpu.VMEM_SHARED((n, d), jnp.float32),
            pltpu.VMEM((_CH,), jnp.int32),
            pltpu.VMEM((_CH,), jnp.int32),
            pltpu.VMEM((_CH,), jnp.int32),
            pltpu.VMEM((_CH,), jnp.int32),
            pltpu.VMEM((_CH, d), jnp.float32),
            pltpu.VMEM((_CH, d), jnp.float32),
            pltpu.VMEM((max(tail, 8),), jnp.int32),
            pltpu.VMEM((max(tail, 8),), jnp.int32),
            pltpu.VMEM((max(tail, 8), d), jnp.float32),
            pltpu.SemaphoreType.DMA,
            pltpu.SemaphoreType.DMA,
            pltpu.SemaphoreType.DMA,
        ],
    )
    def edge_agg(hp_hbm, src_hbm, dst_hbm, zero_hbm, out_hbm,
                 acc, src0, src1, dst0, dst1, rows0, rows1,
                 src_t, dst_t, rows_t, sem0, sem1, sem_s):
        c = lax.axis_index("c")
        s = lax.axis_index("s")
        wid = s * _NC + c
        row0 = pl.multiple_of(s * rpt, 8)
        @pl.when(s < _NS - 1)
        def _():
            pltpu.sync_copy(zero_hbm.at[pl.ds(0, rpt)],
                            acc.at[pl.ds(row0, rpt)])
        @pl.when(s == _NS - 1)
        def _():
            pltpu.sync_copy(zero_hbm, acc.at[pl.ds((_NS - 1) * rpt, rlast)])
        plsc.subcore_barrier()

        base = wid * epw

        def pair(j, carry):
            o0 = base + 2 * j * _CH
            pltpu.sync_copy(src_hbm.at[pl.ds(o0, _CH)], src0)
            ga = pltpu.async_copy(hp_hbm.at[src0], rows0, sem0)
            pltpu.sync_copy(src_hbm.at[pl.ds(o0 + _CH, _CH)], src1)
            gb = pltpu.async_copy(hp_hbm.at[src1], rows1, sem1)
            pltpu.sync_copy(dst_hbm.at[pl.ds(o0, _CH)], dst0)
            pltpu.sync_copy(dst_hbm.at[pl.ds(o0 + _CH, _CH)], dst1)
            ga.wait()
            sa = pltpu.make_async_copy(rows0, acc.at[dst0], sem_s)
            sa.start(add=True)
            gb.wait()
            pltpu.sync_copy(rows1, acc.at[dst1], add=True)
            sa.wait()
            return carry

        lax.fori_loop(0, npair, pair, 0)
        if tail:
            ot = base + npair * 2 * _CH
            pltpu.sync_copy(src_hbm.at[pl.ds(ot, tail)], src_t)
            gt = pltpu.async_copy(hp_hbm.at[src_t], rows_t, sem0)
            pltpu.sync_copy(dst_hbm.at[pl.ds(ot, tail)], dst_t)
            gt.wait()
            pltpu.sync_copy(rows_t, acc.at[dst_t], add=True)
        plsc.subcore_barrier()
        ob = pl.multiple_of(c * n + row0, 8)
        @pl.when(s < _NS - 1)
        def _():
            pltpu.sync_copy(acc.at[pl.ds(row0, rpt)],
                            out_hbm.at[pl.ds(ob, rpt)])
        @pl.when(s == _NS - 1)
        def _():
            pltpu.sync_copy(acc.at[pl.ds((_NS - 1) * rpt, rlast)],
                            out_hbm.at[pl.ds(c * n + (_NS - 1) * rpt, rlast)])

    return edge_agg


def kernel(x, edge_index, batch, W_in, b_in, g_in, bt_in, We, be, ge, bte,
           Wc, gn, btn, W1, b1, g1, bt1, W2, b2):
    n, d = x.shape
    e = edge_index.shape[1]
    nlayers = Wc.shape[0]
    g = 64
    row = lambda v: v.reshape(1, d)

    src = edge_index[0]
    dst = edge_index[1]
    cs = jnp.maximum(bte, 0.0)
    rpt = (n // _NS) // 8 * 8
    zrows = jnp.zeros((n - (_NS - 1) * rpt, d), jnp.float32)

    sds = jax.ShapeDtypeStruct
    two_nd = [sds((n, d), jnp.float32), sds((n, d), jnp.float32)]
    h, hp = pl.pallas_call(_enc_body, out_shape=two_nd)(
        x, W_in, row(b_in), row(g_in), row(bt_in), row(cs[0]))

    edge_agg = _make_edge_agg(n, d, e)
    for i in range(nlayers):
        aggp = edge_agg(hp, src, dst, zrows)
        if i + 1 < nlayers:
            h, hp = pl.pallas_call(_layer_body, out_shape=two_nd)(
                h, aggp, Wc[i], row(gn[i]), row(btn[i]), row(cs[i + 1]))
        else:
            out = pl.pallas_call(
                _final_body, out_shape=sds((g, d), jnp.float32))(
                    h, aggp, Wc[i], row(gn[i]), row(btn[i]),
                    batch.reshape(n, 1), W1, row(b1), row(g1), row(bt1),
                    W2, row(b2))
    return out


# speedup vs baseline: 3.3572x; 1.0522x over previous
import functools

import jax
import jax.numpy as jnp
from jax import lax
from jax.experimental import pallas as pl
from jax.experimental.pallas import tpu as pltpu
from jax.experimental.pallas import tpu_sc as plsc

_EPS = 1e-5
_NC = 2
_NS = 16
_NW = _NC * _NS
_CH = 128


def _bn_relu(y, g, bt):
    m = jnp.mean(y, axis=0, keepdims=True)
    v = jnp.mean((y - m) ** 2, axis=0, keepdims=True)
    return jnp.maximum((y - m) * lax.rsqrt(v + _EPS) * g + bt, 0.0)


def _matT(a, w):
    return lax.dot_general(a, w, (((1,), (1,)), ((), ())),
                           preferred_element_type=jnp.float32)


def _enc_body(x_ref, w_ref, b_ref, g_ref, bt_ref, c_ref, h_ref, hp_ref):
    h = _bn_relu(_matT(x_ref[...], w_ref[...]) + b_ref[...], g_ref[...],
                 bt_ref[...])
    h_ref[...] = h
    hp_ref[...] = jnp.maximum(h + c_ref[...], 0.0)


def _layer_body(h_ref, agg_ref, w_ref, g_ref, bt_ref, c_ref, h_ref_o, hp_ref):
    n = h_ref.shape[0]
    ag = agg_ref[...]
    h = h_ref[...]
    u = h + ag[:n] + ag[n:]
    hn = _bn_relu(_matT(u, w_ref[...]), g_ref[...], bt_ref[...]) + h
    h_ref_o[...] = hn
    hp_ref[...] = jnp.maximum(hn + c_ref[...], 0.0)


def _final_body(h_ref, agg_ref, w_ref, g_ref, bt_ref, batch_ref, w1_ref,
                b1_ref, g1_ref, bt1_ref, w2_ref, b2_ref, out_ref):
    n = h_ref.shape[0]
    g = out_ref.shape[0]
    ag = agg_ref[...]
    h = h_ref[...]
    u = h + ag[:n] + ag[n:]
    hn = _bn_relu(_matT(u, w_ref[...]), g_ref[...], bt_ref[...]) + h
    onehot = (batch_ref[...] == lax.broadcasted_iota(jnp.int32, (n, g), 1)
              ).astype(jnp.float32)
    pooled = lax.dot_general(onehot, hn, (((0,), (0,)), ((), ())),
                             preferred_element_type=jnp.float32,
                             precision=lax.Precision.HIGHEST)
    o = _bn_relu(_matT(pooled, w1_ref[...]) + b1_ref[...], g1_ref[...],
                 bt1_ref[...])
    out_ref[...] = _matT(o, w2_ref[...]) + b2_ref[...]


@functools.lru_cache(maxsize=None)
def _make_edge_agg(n, d, e):
    assert e % _NW == 0 and n % _NS == 0
    epw = e // _NW
    nfull = epw // _CH
    npair = nfull // 2
    tail = epw - npair * 2 * _CH
    assert tail % 8 == 0 and tail <= _CH
    rpt = (n // _NS) // 8 * 8
    rlast = n - (_NS - 1) * rpt
    assert rlast % 8 == 0
    mesh = plsc.VectorSubcoreMesh(core_axis_name="c", subcore_axis_name="s")

    @functools.partial(
        pl.kernel,
        out_type=jax.ShapeDtypeStruct((2 * n, d), jnp.float32),
        mesh=mesh,
        scratch_types=[
            pltpu.VMEM_SHARED((n, d), jnp.float32),
            pltpu.VMEM((_CH,), jnp.int32),
            pltpu.VMEM((_CH,), jnp.int32),
            pltpu.VMEM((_CH,), jnp.int32),
            pltpu.VMEM((_CH,), jnp.int32),
            pltpu.VMEM((_CH, d), jnp.float32),
            pltpu.VMEM((_CH, d), jnp.float32),
            pltpu.VMEM((max(tail, 8),), jnp.int32),
            pltpu.VMEM((max(tail, 8),), jnp.int32),
            pltpu.VMEM((max(tail, 8), d), jnp.float32),
            pltpu.SemaphoreType.DMA,
            pltpu.SemaphoreType.DMA,
            pltpu.SemaphoreType.DMA,
            pltpu.SemaphoreType.DMA,
        ],
    )
    def edge_agg(hp_hbm, src_hbm, dst_hbm, zero_hbm, out_hbm,
                 acc, src0, src1, dst0, dst1, rows0, rows1,
                 src_t, dst_t, rows_t, sem0, sem1, sem_s, sem_i):
        c = lax.axis_index("c")
        s = lax.axis_index("s")
        wid = s * _NC + c
        row0 = pl.multiple_of(s * rpt, 8)
        @pl.when(s < _NS - 1)
        def _():
            pltpu.sync_copy(zero_hbm.at[pl.ds(0, rpt)],
                            acc.at[pl.ds(row0, rpt)])
        @pl.when(s == _NS - 1)
        def _():
            pltpu.sync_copy(zero_hbm, acc.at[pl.ds((_NS - 1) * rpt, rlast)])
        plsc.subcore_barrier()

        base = wid * epw

        def src_loads(j):
            o0 = base + 2 * j * _CH
            return ((src_hbm.at[pl.ds(o0, _CH)], src0),
                    (src_hbm.at[pl.ds(o0 + _CH, _CH)], src1))

        def dst_loads(j):
            o0 = base + 2 * j * _CH
            return ((dst_hbm.at[pl.ds(o0, _CH)], dst0),
                    (dst_hbm.at[pl.ds(o0 + _CH, _CH)], dst1))

        for sref, dref in src_loads(0) + dst_loads(0):
            pltpu.async_copy(sref, dref, sem_i)

        def pair(j, carry):
            for sref, dref in src_loads(j):
                pltpu.make_async_copy(sref, dref, sem_i).wait()
            ga = pltpu.async_copy(hp_hbm.at[src0], rows0, sem0)
            gb = pltpu.async_copy(hp_hbm.at[src1], rows1, sem1)
            ga.wait()
            for sref, dref in dst_loads(j):
                pltpu.make_async_copy(sref, dref, sem_i).wait()
            sa = pltpu.make_async_copy(rows0, acc.at[dst0], sem_s)
            sa.start(add=True)
            gb.wait()
            @pl.when(j + 1 < npair)
            def _():
                for sref, dref in src_loads(j + 1):
                    pltpu.async_copy(sref, dref, sem_i)
            pltpu.sync_copy(rows1, acc.at[dst1], add=True)
            sa.wait()
            @pl.when(j + 1 < npair)
            def _():
                for sref, dref in dst_loads(j + 1):
                    pltpu.async_copy(sref, dref, sem_i)
            return carry

        lax.fori_loop(0, npair, pair, 0)
        if tail:
            ot = base + npair * 2 * _CH
            pltpu.sync_copy(src_hbm.at[pl.ds(ot, tail)], src_t)
            gt = pltpu.async_copy(hp_hbm.at[src_t], rows_t, sem0)
            pltpu.sync_copy(dst_hbm.at[pl.ds(ot, tail)], dst_t)
            gt.wait()
            pltpu.sync_copy(rows_t, acc.at[dst_t], add=True)
        plsc.subcore_barrier()
        ob = pl.multiple_of(c * n + row0, 8)
        @pl.when(s < _NS - 1)
        def _():
            pltpu.sync_copy(acc.at[pl.ds(row0, rpt)],
                            out_hbm.at[pl.ds(ob, rpt)])
        @pl.when(s == _NS - 1)
        def _():
            pltpu.sync_copy(acc.at[pl.ds((_NS - 1) * rpt, rlast)],
                            out_hbm.at[pl.ds(c * n + (_NS - 1) * rpt, rlast)])

    return edge_agg


def kernel(x, edge_index, batch, W_in, b_in, g_in, bt_in, We, be, ge, bte,
           Wc, gn, btn, W1, b1, g1, bt1, W2, b2):
    n, d = x.shape
    e = edge_index.shape[1]
    nlayers = Wc.shape[0]
    g = 64
    row = lambda v: v.reshape(1, d)

    src = edge_index[0]
    dst = edge_index[1]
    cs = jnp.maximum(bte, 0.0)
    rpt = (n // _NS) // 8 * 8
    zrows = jnp.zeros((n - (_NS - 1) * rpt, d), jnp.float32)

    sds = jax.ShapeDtypeStruct
    two_nd = [sds((n, d), jnp.float32), sds((n, d), jnp.float32)]
    h, hp = pl.pallas_call(_enc_body, out_shape=two_nd)(
        x, W_in, row(b_in), row(g_in), row(bt_in), row(cs[0]))

    edge_agg = _make_edge_agg(n, d, e)
    for i in range(nlayers):
        aggp = edge_agg(hp, src, dst, zrows)
        if i + 1 < nlayers:
            h, hp = pl.pallas_call(_layer_body, out_shape=two_nd)(
                h, aggp, Wc[i], row(gn[i]), row(btn[i]), row(cs[i + 1]))
        else:
            out = pl.pallas_call(
                _final_body, out_shape=sds((g, d), jnp.float32))(
                    h, aggp, Wc[i], row(gn[i]), row(btn[i]),
                    batch.reshape(n, 1), W1, row(b1), row(g1), row(bt1),
                    W2, row(b2))
    return out


# speedup vs baseline: 3.7533x; 1.1180x over previous
import functools

import jax
import jax.numpy as jnp
from jax import lax
from jax.experimental import pallas as pl
from jax.experimental.pallas import tpu as pltpu
from jax.experimental.pallas import tpu_sc as plsc

_EPS = 1e-5
_NC = 2
_NS = 16
_NW = _NC * _NS
_CH = 128


def _bn_relu(y, g, bt):
    m = jnp.mean(y, axis=0, keepdims=True)
    v = jnp.mean((y - m) ** 2, axis=0, keepdims=True)
    return jnp.maximum((y - m) * lax.rsqrt(v + _EPS) * g + bt, 0.0)


def _matT(a, w):
    return lax.dot_general(a, w, (((1,), (1,)), ((), ())),
                           preferred_element_type=jnp.float32)


def _enc_body(x_ref, w_ref, b_ref, g_ref, bt_ref, c_ref, h_ref, hp_ref):
    h = _bn_relu(_matT(x_ref[...], w_ref[...]) + b_ref[...], g_ref[...],
                 bt_ref[...])
    h_ref[...] = h
    hp_ref[...] = jnp.maximum(h + c_ref[...], 0.0)


def _layer_body(h_ref, agg_ref, w_ref, g_ref, bt_ref, c_ref, h_ref_o, hp_ref):
    n = h_ref.shape[0]
    ag = agg_ref[...]
    h = h_ref[...]
    u = h + ag[:n] + ag[n:]
    hn = _bn_relu(_matT(u, w_ref[...]), g_ref[...], bt_ref[...]) + h
    h_ref_o[...] = hn
    hp_ref[...] = jnp.maximum(hn + c_ref[...], 0.0)


def _final_body(h_ref, agg_ref, w_ref, g_ref, bt_ref, batch_ref, w1_ref,
                b1_ref, g1_ref, bt1_ref, w2_ref, b2_ref, out_ref):
    n = h_ref.shape[0]
    g = out_ref.shape[0]
    ag = agg_ref[...]
    h = h_ref[...]
    u = h + ag[:n] + ag[n:]
    hn = _bn_relu(_matT(u, w_ref[...]), g_ref[...], bt_ref[...]) + h
    onehot = (batch_ref[...] == lax.broadcasted_iota(jnp.int32, (n, g), 1)
              ).astype(jnp.float32)
    pooled = lax.dot_general(onehot, hn, (((0,), (0,)), ((), ())),
                             preferred_element_type=jnp.float32,
                             precision=lax.Precision.HIGHEST)
    o = _bn_relu(_matT(pooled, w1_ref[...]) + b1_ref[...], g1_ref[...],
                 bt1_ref[...])
    out_ref[...] = _matT(o, w2_ref[...]) + b2_ref[...]


@functools.lru_cache(maxsize=None)
def _make_edge_agg(n, d, e):
    assert e % _NW == 0 and n % _NS == 0
    epw = e // _NW
    nfull = epw // _CH
    assert nfull % 3 == 0
    tail = epw - nfull * _CH
    assert tail % 8 == 0 and tail <= _CH
    rpt = (n // _NS) // 8 * 8
    rlast = n - (_NS - 1) * rpt
    assert rlast % 8 == 0
    mesh = plsc.VectorSubcoreMesh(core_axis_name="c", subcore_axis_name="s")

    @functools.partial(
        pl.kernel,
        out_type=jax.ShapeDtypeStruct((2 * n, d), jnp.float32),
        mesh=mesh,
        scratch_types=[
            pltpu.VMEM_SHARED((n, d), jnp.float32),
            [pltpu.VMEM((_CH,), jnp.int32)] * 3,
            [pltpu.VMEM((_CH,), jnp.int32)] * 3,
            [pltpu.VMEM((_CH, d), jnp.float32)] * 3,
            pltpu.VMEM((max(tail, 8),), jnp.int32),
            pltpu.VMEM((max(tail, 8),), jnp.int32),
            [pltpu.SemaphoreType.DMA] * 3,
            [pltpu.SemaphoreType.DMA] * 3,
            pltpu.SemaphoreType.DMA,
            pltpu.SemaphoreType.DMA,
        ],
    )
    def edge_agg(hp_hbm, src_hbm, dst_hbm, zero_hbm, out_hbm,
                 acc, srcb, dstb, rows, src_t, dst_t,
                 gsem, ssem, sem_is, sem_id):
        c = lax.axis_index("c")
        s = lax.axis_index("s")
        wid = s * _NC + c
        row0 = pl.multiple_of(s * rpt, 8)
        @pl.when(s < _NS - 1)
        def _():
            pltpu.sync_copy(zero_hbm.at[pl.ds(0, rpt)],
                            acc.at[pl.ds(row0, rpt)])
        @pl.when(s == _NS - 1)
        def _():
            pltpu.sync_copy(zero_hbm, acc.at[pl.ds((_NS - 1) * rpt, rlast)])
        plsc.subcore_barrier()

        base = wid * epw

        def load_src(j, p):
            pltpu.async_copy(src_hbm.at[pl.ds(base + j * _CH, _CH)],
                             srcb[p], sem_is)

        def load_dst(j, p):
            pltpu.async_copy(dst_hbm.at[pl.ds(base + j * _CH, _CH)],
                             dstb[p], sem_id)

        def wait_src(p):
            pltpu.make_async_copy(src_hbm.at[pl.ds(base, _CH)], srcb[p],
                                  sem_is).wait()

        def wait_dst(p):
            pltpu.make_async_copy(dst_hbm.at[pl.ds(base, _CH)], dstb[p],
                                  sem_id).wait()

        def wait_scatter(p):
            pltpu.make_async_copy(rows[p], acc.at[dstb[p]], ssem[p]).wait()

        load_src(0, 0)
        wait_src(0)
        pltpu.async_copy(hp_hbm.at[srcb[0]], rows[0], gsem[0])
        load_src(1, 1)
        load_dst(0, 0)

        def group(i, carry):
            for u in range(3):
                p, p1, p2 = u, (u + 1) % 3, (u + 2) % 3
                j = 3 * i + u
                pltpu.make_async_copy(hp_hbm.at[srcb[p]], rows[p],
                                      gsem[p]).wait()
                wait_dst(p)
                pltpu.make_async_copy(rows[p], acc.at[dstb[p]],
                                      ssem[p]).start(add=True)
                @pl.when(j >= 2)
                def _():
                    wait_scatter(p1)
                @pl.when(j + 1 < nfull)
                def _():
                    load_dst(j + 1, p1)
                    wait_src(p1)
                    pltpu.async_copy(hp_hbm.at[srcb[p1]], rows[p1], gsem[p1])
                @pl.when(j + 2 < nfull)
                def _():
                    load_src(j + 2, p2)
            return carry

        lax.fori_loop(0, nfull // 3, group, 0)
        wait_scatter((nfull - 2) % 3)
        wait_scatter((nfull - 1) % 3)
        if tail:
            ot = base + nfull * _CH
            pltpu.sync_copy(src_hbm.at[pl.ds(ot, tail)], src_t)
            gt = pltpu.async_copy(hp_hbm.at[src_t],
                                  rows[0].at[pl.ds(0, tail)], gsem[0])
            pltpu.sync_copy(dst_hbm.at[pl.ds(ot, tail)], dst_t)
            gt.wait()
            pltpu.sync_copy(rows[0].at[pl.ds(0, tail)], acc.at[dst_t],
                            add=True)
        plsc.subcore_barrier()
        ob = pl.multiple_of(c * n + row0, 8)
        @pl.when(s < _NS - 1)
        def _():
            pltpu.sync_copy(acc.at[pl.ds(row0, rpt)],
                            out_hbm.at[pl.ds(ob, rpt)])
        @pl.when(s == _NS - 1)
        def _():
            pltpu.sync_copy(acc.at[pl.ds((_NS - 1) * rpt, rlast)],
                            out_hbm.at[pl.ds(c * n + (_NS - 1) * rpt, rlast)])

    return edge_agg


def kernel(x, edge_index, batch, W_in, b_in, g_in, bt_in, We, be, ge, bte,
           Wc, gn, btn, W1, b1, g1, bt1, W2, b2):
    n, d = x.shape
    e = edge_index.shape[1]
    nlayers = Wc.shape[0]
    g = 64
    row = lambda v: v.reshape(1, d)

    src = edge_index[0]
    dst = edge_index[1]
    cs = jnp.maximum(bte, 0.0)
    rpt = (n // _NS) // 8 * 8
    zrows = jnp.zeros((n - (_NS - 1) * rpt, d), jnp.float32)

    sds = jax.ShapeDtypeStruct
    two_nd = [sds((n, d), jnp.float32), sds((n, d), jnp.float32)]
    h, hp = pl.pallas_call(_enc_body, out_shape=two_nd)(
        x, W_in, row(b_in), row(g_in), row(bt_in), row(cs[0]))

    edge_agg = _make_edge_agg(n, d, e)
    for i in range(nlayers):
        aggp = edge_agg(hp, src, dst, zrows)
        if i + 1 < nlayers:
            h, hp = pl.pallas_call(_layer_body, out_shape=two_nd)(
                h, aggp, Wc[i], row(gn[i]), row(btn[i]), row(cs[i + 1]))
        else:
            out = pl.pallas_call(
                _final_body, out_shape=sds((g, d), jnp.float32))(
                    h, aggp, Wc[i], row(gn[i]), row(btn[i]),
                    batch.reshape(n, 1), W1, row(b1), row(g1), row(bt1),
                    W2, row(b2))
    return out
